# speedup vs baseline: 2.8235x; 2.8235x over previous
import functools

import jax
import jax.numpy as jnp
from jax import lax
from jax.experimental import pallas as pl
from jax.experimental.pallas import tpu as pltpu
from jax.experimental.pallas import tpu_sc as plsc

_NC, _NS = 2, 16
_NW = _NC * _NS
_B, _NEG, _D = 1024, 200, 128
_TAIL = _B * (_NEG + 1)
_RPW = _TAIL // _NW
_CH = 96
_NCHUNK = _RPW // _CH
_HPW = _B // _NW


def _sc_gather(entity, relation, head_idx, rel_idx, tail_idx):
    mesh = plsc.VectorSubcoreMesh(core_axis_name="c", subcore_axis_name="s")

    @functools.partial(
        pl.kernel,
        mesh=mesh,
        out_type=[
            jax.ShapeDtypeStruct((_B, _D), jnp.float32),
            jax.ShapeDtypeStruct((_B, _D), jnp.float32),
            jax.ShapeDtypeStruct((_TAIL, _D), jnp.float32),
        ],
        scratch_types=[
            pltpu.VMEM((_HPW,), jnp.int32),
            pltpu.VMEM((_HPW, _D), jnp.float32),
            pltpu.VMEM((_NCHUNK, _CH), jnp.int32),
            pltpu.VMEM((_CH, _D), jnp.float32),
            pltpu.SemaphoreType.DMA,
        ],
    )
    def k(ent_hbm, rel_hbm, hidx_hbm, ridx_hbm, tidx_hbm,
          head_out, rel_out, tail_out,
          sidx_v, srow_v, tidx_v, trow_v, sem):
        wid = lax.axis_index("s") * _NC + lax.axis_index("c")

        hbase = wid * _HPW
        pltpu.sync_copy(hidx_hbm.at[wid], sidx_v)
        pltpu.async_copy(ent_hbm.at[sidx_v], srow_v, sem).wait()
        pltpu.sync_copy(srow_v, head_out.at[pl.ds(hbase, _HPW)])

        pltpu.sync_copy(ridx_hbm.at[wid], sidx_v)
        pltpu.async_copy(rel_hbm.at[sidx_v], srow_v, sem).wait()
        pltpu.sync_copy(srow_v, rel_out.at[pl.ds(hbase, _HPW)])

        tbase = wid * _RPW
        pltpu.sync_copy(tidx_hbm.at[wid], tidx_v)

        def body(j, carry):
            pltpu.async_copy(ent_hbm.at[tidx_v.at[j]], trow_v, sem).wait()
            pltpu.sync_copy(trow_v, tail_out.at[pl.ds(tbase + j * _CH, _CH)])
            return carry

        lax.fori_loop(0, _NCHUNK, body, 0)

    return k(entity, relation, head_idx, rel_idx, tail_idx)


def kernel(positive, negative, entity_embedding, relation_embedding):
    positive = positive.astype(jnp.int32)
    negative = negative.astype(jnp.int32)
    head_idx = positive[:, 0].reshape(_NW, _HPW)
    rel_idx = positive[:, 1].reshape(_NW, _HPW)
    tail_idx = jnp.concatenate(
        [positive[:, 2:3], negative], axis=1).reshape(_NW, _NCHUNK, _CH)
    head, rel, tail = _sc_gather(
        entity_embedding, relation_embedding, head_idx, rel_idx, tail_idx)
    return (head[:, None, :], rel[:, None, :], tail.reshape(_B, _NEG + 1, _D))
